# Initial kernel scaffold; baseline (speedup 1.0000x reference)
import jax
import jax.numpy as jnp
from jax.experimental import pallas as pl


def kernel(x, x_len, mask, emb_weight, W1, b1, W2, b2):
    raise NotImplementedError("write your pallas kernel here")


# speedup vs baseline: 4.7775x; 4.7775x over previous
import functools

import jax
import jax.numpy as jnp
import numpy as np
from jax import lax
from jax.experimental import pallas as pl
from jax.experimental.pallas import tpu as pltpu
from jax.experimental.pallas import tpu_sc as plsc

B = 4096
L = 200
VOCAB = 100000
D = 100
H = 50
HP = 64
NUM_CLASSES = 20
PAD = 1

NC, NS = 2, 16
NW = NC * NS
BPW = B // NW
TPW = BPW * L
CHUNK = 128
NCHUNK = TPW // CHUNK


def _fold_body(emb_ref, w1p_ref, out_ref):
    out_ref[...] = jnp.dot(emb_ref[...], w1p_ref[...],
                           preferred_element_type=jnp.float32)


def _fold_table(emb, w1p):
    blk = 2000
    return pl.pallas_call(
        _fold_body,
        grid=(VOCAB // blk,),
        in_specs=[
            pl.BlockSpec((blk, D), lambda i: (i, 0)),
            pl.BlockSpec((D, HP), lambda i: (0, 0)),
        ],
        out_specs=pl.BlockSpec((blk, HP), lambda i: (i, 0)),
        out_shape=jax.ShapeDtypeStruct((VOCAB, HP), jnp.float32),
    )(emb, w1p)


def _mask_body(x_ref, m_ref, out_ref):
    out_ref[...] = jnp.where(m_ref[...], x_ref[...], PAD)


def _mask_indices(x, mask):
    blk = 512
    return pl.pallas_call(
        _mask_body,
        grid=(B // blk,),
        in_specs=[
            pl.BlockSpec((blk, L), lambda i: (i, 0)),
            pl.BlockSpec((blk, L), lambda i: (i, 0)),
        ],
        out_specs=pl.BlockSpec((blk, L), lambda i: (i, 0)),
        out_shape=jax.ShapeDtypeStruct((B, L), jnp.int32),
    )(x, mask)


def _pool_body(table_hbm, xm_hbm, dest_hbm, zeros_hbm, out_hbm,
               idx_v, dest_v, rows_v, acc_sh):
    c = lax.axis_index("c")
    s = lax.axis_index("s")
    base_b = c * (NS * BPW) + s * BPW
    base_t = base_b * L
    slot0 = s * BPW

    pltpu.sync_copy(zeros_hbm, acc_sh.at[pl.ds(slot0, BPW)])

    @pl.loop(0, NCHUNK)
    def _(i):
        off = i * CHUNK
        pltpu.sync_copy(xm_hbm.at[pl.ds(base_t + off, CHUNK)], idx_v)
        pltpu.sync_copy(dest_hbm.at[pl.ds(off, CHUNK)], dest_v)
        for k in range(CHUNK // 16):
            dest_v[pl.ds(k * 16, 16)] = dest_v[pl.ds(k * 16, 16)] + slot0
        pltpu.sync_copy(table_hbm.at[idx_v], rows_v)
        pltpu.sync_copy(rows_v, acc_sh.at[dest_v], add=True)

    pltpu.sync_copy(acc_sh.at[pl.ds(slot0, BPW)], out_hbm.at[pl.ds(base_b, BPW)])


@functools.partial(jax.jit, static_argnums=())
def _pool(table, xm_flat, dest, zeros):
    mesh = plsc.VectorSubcoreMesh(core_axis_name="c", subcore_axis_name="s")
    kern = pl.kernel(
        _pool_body,
        out_type=jax.ShapeDtypeStruct((B, HP), jnp.float32),
        mesh=mesh,
        compiler_params=pltpu.CompilerParams(use_tc_tiling_on_sc=False),
        scratch_types=[
            pltpu.VMEM((CHUNK,), jnp.int32),
            pltpu.VMEM((CHUNK,), jnp.int32),
            pltpu.VMEM((CHUNK, HP), jnp.float32),
            pltpu.VMEM_SHARED((NS * BPW, HP), jnp.float32),
        ],
    )
    return kern(table, xm_flat, dest, zeros)


def _tail_body(acc_ref, len_ref, b1p_ref, w2p_ref, b2_ref, out_ref):
    pooled = acc_ref[...] / len_ref[...]
    h = jnp.maximum(pooled + b1p_ref[...], 0.0)
    out_ref[...] = (jnp.dot(h, w2p_ref[...], preferred_element_type=jnp.float32)
                    + b2_ref[...])


def _tail(acc, len_f, b1p, w2p, b2):
    blk = 512
    return pl.pallas_call(
        _tail_body,
        grid=(B // blk,),
        in_specs=[
            pl.BlockSpec((blk, HP), lambda i: (i, 0)),
            pl.BlockSpec((blk, 1), lambda i: (i, 0)),
            pl.BlockSpec((1, HP), lambda i: (0, 0)),
            pl.BlockSpec((HP, NUM_CLASSES), lambda i: (0, 0)),
            pl.BlockSpec((1, NUM_CLASSES), lambda i: (0, 0)),
        ],
        out_specs=pl.BlockSpec((blk, NUM_CLASSES), lambda i: (i, 0)),
        out_shape=jax.ShapeDtypeStruct((B, NUM_CLASSES), jnp.float32),
    )(acc, len_f, b1p, w2p, b2)


_DEST = jnp.asarray(np.repeat(np.arange(BPW, dtype=np.int32), L))
_ZEROS = jnp.zeros((BPW, HP), jnp.float32)


def kernel(x, x_len, mask, emb_weight, W1, b1, W2, b2):
    w1p = jnp.pad(W1.T, ((0, 0), (0, HP - H)))
    table = _fold_table(emb_weight, w1p)
    xm = _mask_indices(x, mask).reshape(B * L)
    acc = _pool(table, xm, _DEST, _ZEROS)
    len_f = x_len.astype(jnp.float32).reshape(B, 1)
    b1p = jnp.pad(b1, (0, HP - H)).reshape(1, HP)
    w2p = jnp.pad(W2.T, ((0, HP - H), (0, 0)))
    b2r = b2.reshape(1, NUM_CLASSES)
    return _tail(acc, len_f, b1p, w2p, b2r)


# speedup vs baseline: 7.3853x; 1.5459x over previous
import functools

import jax
import jax.numpy as jnp
import numpy as np
from jax import lax
from jax.experimental import pallas as pl
from jax.experimental.pallas import tpu as pltpu
from jax.experimental.pallas import tpu_sc as plsc

B = 4096
L = 200
VOCAB = 100000
D = 100
H = 50
HP = 64
NUM_CLASSES = 20
PAD = 1

NC, NS = 2, 16
NW = NC * NS
BPW = B // NW
TPW = BPW * L
CHUNK = 128
NCHUNK = TPW // CHUNK


def _fold_body(emb_ref, w1p_ref, out_ref):
    out_ref[...] = jnp.dot(emb_ref[...], w1p_ref[...],
                           preferred_element_type=jnp.float32)


def _fold_table(emb, w1p):
    blk = 2000
    return pl.pallas_call(
        _fold_body,
        grid=(VOCAB // blk,),
        in_specs=[
            pl.BlockSpec((blk, D), lambda i: (i, 0)),
            pl.BlockSpec((D, HP), lambda i: (0, 0)),
        ],
        out_specs=pl.BlockSpec((blk, HP), lambda i: (i, 0)),
        out_shape=jax.ShapeDtypeStruct((VOCAB, HP), jnp.float32),
    )(emb, w1p)


def _mask_body(x_ref, m_ref, out_ref):
    out_ref[...] = jnp.where(m_ref[...], x_ref[...], PAD)


def _mask_indices(x, mask):
    blk = 512
    return pl.pallas_call(
        _mask_body,
        grid=(B // blk,),
        in_specs=[
            pl.BlockSpec((blk, L), lambda i: (i, 0)),
            pl.BlockSpec((blk, L), lambda i: (i, 0)),
        ],
        out_specs=pl.BlockSpec((blk, L), lambda i: (i, 0)),
        out_shape=jax.ShapeDtypeStruct((B, L), jnp.int32),
    )(x, mask)


NBUF = 4


def _pool_body(table_hbm, xm_hbm, dest_hbm, zeros_hbm, out_hbm,
               dest_v, idx_c0, idx_c1, idx_c2, idx_c3,
               dest_c0, dest_c1, dest_c2, dest_c3,
               rows_v, acc_sh,
               isem0, isem1, isem2, isem3,
               gsem0, gsem1, gsem2, gsem3, ssem0, ssem1, ssem2, ssem3):
    idx_cs = [idx_c0, idx_c1, idx_c2, idx_c3]
    dest_cs = [dest_c0, dest_c1, dest_c2, dest_c3]
    isems = [isem0, isem1, isem2, isem3]
    gsems = [gsem0, gsem1, gsem2, gsem3]
    ssems = [ssem0, ssem1, ssem2, ssem3]
    c = lax.axis_index("c")
    s = lax.axis_index("s")
    tile = c * NS + s
    base_b = tile * BPW
    slot0 = s * BPW

    pltpu.sync_copy(zeros_hbm, acc_sh.at[pl.ds(slot0, BPW)])
    pltpu.sync_copy(dest_hbm, dest_v)

    @pl.loop(0, NCHUNK)
    def _(r):
        for k in range(CHUNK // 16):
            dest_v[r, pl.ds(k * 16, 16)] = dest_v[r, pl.ds(k * 16, 16)] + slot0

    def fire_idx(i, j):
        pltpu.async_copy(xm_hbm.at[tile, i], idx_cs[j], isems[j])

    def wait_idx(i, j):
        pltpu.make_async_copy(xm_hbm.at[tile, i], idx_cs[j], isems[j]).wait()

    def fire_gather(i, j):
        pltpu.async_copy(table_hbm.at[idx_cs[j]], rows_v.at[j], gsems[j])

    def wait_gather(i, j):
        pltpu.make_async_copy(table_hbm.at[idx_cs[j]], rows_v.at[j],
                              gsems[j]).wait()

    def fire_scatter(i, j):
        for k in range(CHUNK // 16):
            dest_cs[j][pl.ds(k * 16, 16)] = dest_v[i, pl.ds(k * 16, 16)]
        pltpu.async_copy(rows_v.at[j], acc_sh.at[dest_cs[j]], ssems[j],
                         add=True)

    def wait_scatter(i, j):
        pltpu.make_async_copy(rows_v.at[j], acc_sh.at[dest_cs[j]],
                              ssems[j]).wait()

    @pl.loop(0, NCHUNK, step=NBUF)
    def _(i0):
        for j in range(NBUF):
            fire_idx(i0 + j, j)
        for j in range(NBUF):
            wait_idx(i0 + j, j)
            fire_gather(i0 + j, j)
        for j in range(NBUF):
            wait_gather(i0 + j, j)
        for j in range(NBUF):
            fire_scatter(i0 + j, j)
            wait_scatter(i0 + j, j)

    pltpu.sync_copy(acc_sh.at[pl.ds(slot0, BPW)], out_hbm.at[pl.ds(base_b, BPW)])


@functools.partial(jax.jit, static_argnums=())
def _pool(table, xm_flat, dest, zeros):
    mesh = plsc.VectorSubcoreMesh(core_axis_name="c", subcore_axis_name="s")
    kern = pl.kernel(
        _pool_body,
        out_type=jax.ShapeDtypeStruct((B, HP), jnp.float32),
        mesh=mesh,
        compiler_params=pltpu.CompilerParams(use_tc_tiling_on_sc=False),
        scratch_types=[
            pltpu.VMEM((NCHUNK, CHUNK), jnp.int32),
            pltpu.VMEM((CHUNK,), jnp.int32),
            pltpu.VMEM((CHUNK,), jnp.int32),
            pltpu.VMEM((CHUNK,), jnp.int32),
            pltpu.VMEM((CHUNK,), jnp.int32),
            pltpu.VMEM((CHUNK,), jnp.int32),
            pltpu.VMEM((CHUNK,), jnp.int32),
            pltpu.VMEM((CHUNK,), jnp.int32),
            pltpu.VMEM((CHUNK,), jnp.int32),
            pltpu.VMEM((NBUF, CHUNK, HP), jnp.float32),
            pltpu.VMEM_SHARED((NS * BPW, HP), jnp.float32),
            pltpu.SemaphoreType.DMA,
            pltpu.SemaphoreType.DMA,
            pltpu.SemaphoreType.DMA,
            pltpu.SemaphoreType.DMA,
            pltpu.SemaphoreType.DMA,
            pltpu.SemaphoreType.DMA,
            pltpu.SemaphoreType.DMA,
            pltpu.SemaphoreType.DMA,
            pltpu.SemaphoreType.DMA,
            pltpu.SemaphoreType.DMA,
            pltpu.SemaphoreType.DMA,
            pltpu.SemaphoreType.DMA,
        ],
    )
    return kern(table, xm_flat, dest, zeros)


def _tail_body(acc_ref, len_ref, b1p_ref, w2p_ref, b2_ref, out_ref):
    pooled = acc_ref[...] / len_ref[...]
    h = jnp.maximum(pooled + b1p_ref[...], 0.0)
    out_ref[...] = (jnp.dot(h, w2p_ref[...], preferred_element_type=jnp.float32)
                    + b2_ref[...])


def _tail(acc, len_f, b1p, w2p, b2):
    blk = 512
    return pl.pallas_call(
        _tail_body,
        grid=(B // blk,),
        in_specs=[
            pl.BlockSpec((blk, HP), lambda i: (i, 0)),
            pl.BlockSpec((blk, 1), lambda i: (i, 0)),
            pl.BlockSpec((1, HP), lambda i: (0, 0)),
            pl.BlockSpec((HP, NUM_CLASSES), lambda i: (0, 0)),
            pl.BlockSpec((1, NUM_CLASSES), lambda i: (0, 0)),
        ],
        out_specs=pl.BlockSpec((blk, NUM_CLASSES), lambda i: (i, 0)),
        out_shape=jax.ShapeDtypeStruct((B, NUM_CLASSES), jnp.float32),
    )(acc, len_f, b1p, w2p, b2)


_DEST = jnp.asarray(
    np.repeat(np.arange(BPW, dtype=np.int32), L).reshape(NCHUNK, CHUNK))
_ZEROS = jnp.zeros((BPW, HP), jnp.float32)


def kernel(x, x_len, mask, emb_weight, W1, b1, W2, b2):
    w1p = jnp.pad(W1.T, ((0, 0), (0, HP - H)))
    table = _fold_table(emb_weight, w1p)
    xm = _mask_indices(x, mask).reshape(NW, NCHUNK, CHUNK)
    acc = _pool(table, xm, _DEST, _ZEROS)
    len_f = x_len.astype(jnp.float32).reshape(B, 1)
    b1p = jnp.pad(b1, (0, HP - H)).reshape(1, HP)
    w2p = jnp.pad(W2.T, ((0, HP - H), (0, 0)))
    b2r = b2.reshape(1, NUM_CLASSES)
    return _tail(acc, len_f, b1p, w2p, b2r)
